# baseline (speedup 1.0000x reference)
import jax
import jax.numpy as jnp
from jax.experimental import pallas as pl

C = 384


def _sigmoid_body(x_ref, o_ref):
    o_ref[...] = jax.nn.sigmoid(x_ref[...])


def kernel(support_map, context_vec, dw_w, dw_b, pw_w, pw_b):
    m = jnp.mean(support_map, axis=0, keepdims=True)
    dn = ('NCHW', 'OIHW', 'NCHW')
    y = jax.lax.conv_general_dilated(m, dw_w, (1, 1), 'SAME',
                                     dimension_numbers=dn,
                                     feature_group_count=C)
    y = y + dw_b[None, :, None, None]
    y = jax.nn.relu(y)
    y = jax.lax.conv_general_dilated(y, pw_w, (1, 1), 'SAME',
                                     dimension_numbers=dn)
    logits = y + pw_b[None, :, None, None]
    x2 = logits.reshape(-1, 128)
    p2 = pl.pallas_call(
        _sigmoid_body,
        out_shape=jax.ShapeDtypeStruct(x2.shape, x2.dtype),
        grid=(x2.shape[0] // 1024,),
        in_specs=[pl.BlockSpec((1024, 128), lambda i: (i, 0))],
        out_specs=pl.BlockSpec((1024, 128), lambda i: (i, 0)),
    )(x2)
    p_flat = p2.reshape(1, -1)
    k = 100000
    vals, idx = jax.lax.top_k(p_flat[0], k)
    mask = jnp.zeros_like(p_flat).at[0, idx].set(1.0)
    return (mask, p_flat)


# speedup vs baseline: 15.9856x; 15.9856x over previous
import functools

import jax
import jax.numpy as jnp
from jax import lax
from jax.experimental import pallas as pl
from jax.experimental.pallas import tpu as pltpu
from jax.experimental.pallas import tpu_sc as plsc

C = 384
H = 224
W = 224
KTOP = 100000
N = C * H * W
NW = 32
CHUNK = N // NW
PIECE = 12288
NPIECE = CHUNK // PIECE
CB = 16
HWT = 512
NROWS = N // 128
MROWS = 1024


def _dw_body(m_ref, w_ref, b_ref, o_ref):
    m = m_ref[...].astype(jnp.float32)
    w = w_ref[...]
    acc = jnp.zeros((CB, H, W), jnp.float32)
    for ki in range(3):
        for kj in range(3):
            wv = w[:, ki, kj][:, None, None]
            acc = acc + wv * m[:, ki:ki + H, kj:kj + W]
    o_ref[...] = jnp.maximum(acc + b_ref[...], 0.0)


def _pw_body(w_ref, y_ref, b_ref, o_ref):
    lo = jnp.dot(w_ref[...].astype(jnp.bfloat16),
                 y_ref[...].astype(jnp.bfloat16),
                 preferred_element_type=jnp.float32)
    o_ref[...] = jax.nn.sigmoid(lo + b_ref[...])


def _wid():
    return lax.axis_index("s") * 2 + lax.axis_index("c")


def _zero_hist(hist, nbins16):
    def z(i, _):
        hist[pl.ds(i * 16, 16)] = jnp.zeros((16,), jnp.int32)
        return 0
    lax.fori_loop(0, nbins16, z, 0)


def _reduce_hist(hist, histr, nbins, nbins16):
    def red(i, _):
        acc = hist[pl.ds(i * 16, 16)]
        for l in range(1, 16):
            acc = acc + hist[pl.ds(l * nbins + i * 16, 16)]
        histr[pl.ds(i * 16, 16)] = acc
        return 0
    lax.fori_loop(0, nbins16, red, 0)


def _stream_chunks(p_hbm, buf, base, per_vec):
    def piece(j, _):
        pltpu.sync_copy(p_hbm.at[pl.ds(base + j * PIECE, PIECE)], buf)
        def vec(v, _):
            per_vec(buf[pl.ds(v * 16, 16)])
            return 0
        lax.fori_loop(0, PIECE // 16, vec, 0, unroll=8)
        return 0
    lax.fori_loop(0, NPIECE, piece, 0)


def _sc_hist1(p_hbm, out_hbm, buf, hist, histr):
    wid = _wid()
    _zero_hist(hist, 4096)
    iota = lax.iota(jnp.int32, 16)
    ones = jnp.ones((16,), jnp.int32)

    def per_vec(key):
        b = key >> 20
        plsc.addupdate_scatter(hist, [iota * 4096 + b], ones)

    _stream_chunks(p_hbm, buf, wid * CHUNK, per_vec)
    _reduce_hist(hist, histr, 4096, 256)
    pltpu.sync_copy(histr, out_hbm.at[wid])


def _sc_hist2(p_hbm, sel_hbm, out_hbm, buf, hist, histr, selv):
    wid = _wid()
    _zero_hist(hist, 4096)
    pltpu.sync_copy(sel_hbm, selv)
    b1 = selv[0]
    iota = lax.iota(jnp.int32, 16)
    ones = jnp.ones((16,), jnp.int32)

    def per_vec(key):
        match = (key >> 20) == b1
        b = (key >> 8) & 0xFFF
        plsc.addupdate_scatter(hist, [iota * 4096 + b], ones, mask=match)

    _stream_chunks(p_hbm, buf, wid * CHUNK, per_vec)
    _reduce_hist(hist, histr, 4096, 256)
    pltpu.sync_copy(histr, out_hbm.at[wid])


def _sc_hist3(p_hbm, sel_hbm, out_hbm, buf, hist, histr, selv):
    wid = _wid()
    _zero_hist(hist, 256)
    pltpu.sync_copy(sel_hbm, selv)
    pref = selv[0]
    iota = lax.iota(jnp.int32, 16)
    ones = jnp.ones((16,), jnp.int32)

    def per_vec(key):
        match = (key >> 8) == pref
        b = key & 0xFF
        plsc.addupdate_scatter(hist, [iota * 256 + b], ones, mask=match)

    _stream_chunks(p_hbm, buf, wid * CHUNK, per_vec)
    _reduce_hist(hist, histr, 256, 16)
    pltpu.sync_copy(histr, out_hbm.at[wid])


@functools.cache
def _sc_kernels():
    mesh = plsc.VectorSubcoreMesh(core_axis_name="c", subcore_axis_name="s")
    cp = pltpu.CompilerParams(needs_layout_passes=False)
    hist1 = pl.kernel(
        _sc_hist1, mesh=mesh, compiler_params=cp,
        out_type=jax.ShapeDtypeStruct((NW, 4096), jnp.int32),
        scratch_types=[pltpu.VMEM((PIECE,), jnp.int32),
                       pltpu.VMEM((4096 * 16,), jnp.int32),
                       pltpu.VMEM((4096,), jnp.int32)])
    hist2 = pl.kernel(
        _sc_hist2, mesh=mesh, compiler_params=cp,
        out_type=jax.ShapeDtypeStruct((NW, 4096), jnp.int32),
        scratch_types=[pltpu.VMEM((PIECE,), jnp.int32),
                       pltpu.VMEM((4096 * 16,), jnp.int32),
                       pltpu.VMEM((4096,), jnp.int32),
                       pltpu.VMEM((2, 16), jnp.int32)])
    hist3 = pl.kernel(
        _sc_hist3, mesh=mesh, compiler_params=cp,
        out_type=jax.ShapeDtypeStruct((NW, 256), jnp.int32),
        scratch_types=[pltpu.VMEM((PIECE,), jnp.int32),
                       pltpu.VMEM((256 * 16,), jnp.int32),
                       pltpu.VMEM((256,), jnp.int32),
                       pltpu.VMEM((2, 16), jnp.int32)])
    return hist1, hist2, hist3


def _cumsum_last(x):
    n = x.shape[-1]
    s = 1
    while s < n:
        shifted = jnp.concatenate(
            [jnp.zeros_like(x[..., :s]), x[..., :n - s]], axis=-1)
        x = x + shifted
        s *= 2
    return x


def _cumsum_rows(x):
    n = x.shape[0]
    s = 1
    while s < n:
        shifted = jnp.concatenate(
            [jnp.zeros_like(x[:s]), x[:n - s]], axis=0)
        x = x + shifted
        s *= 2
    return x


def _find_bin(g, want):
    B = g.shape[1]
    cs = _cumsum_last(g)
    t = jnp.sum(g) - (cs - g)
    iota = lax.broadcasted_iota(jnp.int32, (1, B), 1)
    b = jnp.max(jnp.where(t >= want, iota, -1))
    gb = jnp.sum(jnp.where(iota == b, g, 0))
    tb = jnp.sum(jnp.where(iota == b, t, 0))
    r = want - (tb - gb)
    return b, r


def _red1_body(h_ref, o_ref):
    g = jnp.sum(h_ref[...], axis=0, keepdims=True)
    b1, r1 = _find_bin(g, KTOP)
    o_ref[0, :] = jnp.full((16,), b1, jnp.int32)
    o_ref[1, :] = jnp.full((16,), r1, jnp.int32)


def _red2_body(h_ref, s_ref, o_ref):
    g = jnp.sum(h_ref[...], axis=0, keepdims=True)
    b1 = jnp.max(s_ref[0:1, :])
    r1 = jnp.max(s_ref[1:2, :])
    b2, r2 = _find_bin(g, r1)
    o_ref[0, :] = jnp.full((16,), b1 * 4096 + b2, jnp.int32)
    o_ref[1, :] = jnp.full((16,), r2, jnp.int32)


def _red3_body(h_ref, s_ref, t_ref, r_ref):
    g = jnp.sum(h_ref[...], axis=0, keepdims=True)
    pref = jnp.max(s_ref[0:1, :])
    r2 = jnp.max(s_ref[1:2, :])
    b3, r3 = _find_bin(g, r2)
    tbits = jnp.full((1, 1), pref * 256 + b3, jnp.int32)
    t_ref[...] = lax.bitcast_convert_type(tbits, jnp.float32)
    r_ref[...] = jnp.full((1, 1), r3, jnp.int32)


def _mask_body(t_ref, r_ref, p_ref, o_ref, carry):
    pid = pl.program_id(0)

    @pl.when(pid == 0)
    def _():
        carry[0] = 0

    t = t_ref[0, 0]
    r = r_ref[0, 0]
    p = p_ref[...]
    gt = p > t
    eq = p == t
    eqi = eq.astype(jnp.int32)
    blk = jnp.sum(eqi)
    c0 = carry[0]
    take_all = (c0 + blk) <= r
    o_ref[...] = jnp.where(gt | (eq & take_all), 1.0, 0.0)

    boundary = (c0 < r) & ((c0 + blk) > r)

    @pl.when(boundary)
    def _():
        lane_cum = _cumsum_last(eqi)
        row_tot = jnp.sum(eqi, axis=1, keepdims=True)
        row_cum_excl = _cumsum_rows(row_tot) - row_tot
        rank = c0 + row_cum_excl + lane_cum
        sel = eq & (rank <= r)
        o_ref[...] = jnp.where(gt | sel, 1.0, 0.0)

    carry[0] = c0 + blk


def kernel(support_map, context_vec, dw_w, dw_b, pw_w, pw_b):
    m = jnp.mean(support_map, axis=0)
    m_bf = jnp.pad(m, ((0, 0), (1, 1), (1, 1))).astype(jnp.bfloat16)
    w3 = dw_w.reshape(C, 3, 3)
    b3 = dw_b.reshape(C, 1, 1)

    y = pl.pallas_call(
        _dw_body,
        out_shape=jax.ShapeDtypeStruct((C, H, W), jnp.float32),
        grid=(C // CB,),
        in_specs=[
            pl.BlockSpec((CB, H + 2, W + 2), lambda i: (i, 0, 0)),
            pl.BlockSpec((CB, 3, 3), lambda i: (i, 0, 0)),
            pl.BlockSpec((CB, 1, 1), lambda i: (i, 0, 0)),
        ],
        out_specs=pl.BlockSpec((CB, H, W), lambda i: (i, 0, 0)),
    )(m_bf, w3, b3)

    y2 = y.reshape(C, H * W)
    w2 = pw_w.reshape(C, C)
    b2 = pw_b.reshape(C, 1)

    p2 = pl.pallas_call(
        _pw_body,
        out_shape=jax.ShapeDtypeStruct((C, H * W), jnp.float32),
        grid=(H * W // HWT,),
        in_specs=[
            pl.BlockSpec((C, C), lambda j: (0, 0)),
            pl.BlockSpec((C, HWT), lambda j: (0, j)),
            pl.BlockSpec((C, 1), lambda j: (0, 0)),
        ],
        out_specs=pl.BlockSpec((C, HWT), lambda j: (0, j)),
    )(w2, y2, b2)

    p_lin = p2.reshape(N)
    p_bits = lax.bitcast_convert_type(p_lin, jnp.int32)

    _hist1, _hist2, _hist3 = _sc_kernels()
    h1 = _hist1(p_bits)
    sel1 = pl.pallas_call(
        _red1_body,
        out_shape=jax.ShapeDtypeStruct((2, 16), jnp.int32),
    )(h1)

    h2 = _hist2(p_bits, sel1)
    sel2 = pl.pallas_call(
        _red2_body,
        out_shape=jax.ShapeDtypeStruct((2, 16), jnp.int32),
    )(h2, sel1)

    h3 = _hist3(p_bits, sel2)
    t_arr, r_arr = pl.pallas_call(
        _red3_body,
        out_shape=(jax.ShapeDtypeStruct((1, 1), jnp.float32),
                   jax.ShapeDtypeStruct((1, 1), jnp.int32)),
    )(h3, sel2)

    p_rows = p_lin.reshape(NROWS, 128)
    mask_rows = pl.pallas_call(
        _mask_body,
        out_shape=jax.ShapeDtypeStruct((NROWS, 128), jnp.float32),
        grid=(NROWS // MROWS,),
        in_specs=[
            pl.BlockSpec(memory_space=pltpu.SMEM),
            pl.BlockSpec(memory_space=pltpu.SMEM),
            pl.BlockSpec((MROWS, 128), lambda i: (i, 0)),
        ],
        out_specs=pl.BlockSpec((MROWS, 128), lambda i: (i, 0)),
        scratch_shapes=[pltpu.SMEM((1,), jnp.int32)],
    )(t_arr, r_arr, p_rows)

    return (mask_rows.reshape(1, N), p_lin.reshape(1, N))


# speedup vs baseline: 17.2170x; 1.0770x over previous
import functools

import jax
import jax.numpy as jnp
from jax import lax
from jax.experimental import pallas as pl
from jax.experimental.pallas import tpu as pltpu
from jax.experimental.pallas import tpu_sc as plsc

C = 384
H = 224
W = 224
KTOP = 100000
N = C * H * W
NW = 32
CHUNK = N // NW
PIECE = 25088
NPIECE = CHUNK // PIECE
CB = 16
HWT = 512
NROWS = N // 128
MROWS = 1024


def _dw_body(m_ref, w_ref, b_ref, o_ref):
    m = m_ref[...].astype(jnp.float32)
    w = w_ref[...]
    acc = jnp.zeros((CB, H, W), jnp.float32)
    for ki in range(3):
        for kj in range(3):
            wv = w[:, ki, kj][:, None, None]
            acc = acc + wv * m[:, ki:ki + H, kj:kj + W]
    o_ref[...] = jnp.maximum(acc + b_ref[...], 0.0)


def _pw_body(w_ref, y_ref, b_ref, o_ref):
    lo = jnp.dot(w_ref[...].astype(jnp.bfloat16),
                 y_ref[...].astype(jnp.bfloat16),
                 preferred_element_type=jnp.float32)
    o_ref[...] = jax.nn.sigmoid(lo + b_ref[...])


def _wid():
    return lax.axis_index("s") * 2 + lax.axis_index("c")


def _zero_hist(hist, nbins16):
    def z(i, _):
        hist[pl.ds(i * 16, 16)] = jnp.zeros((16,), jnp.int32)
        return 0
    lax.fori_loop(0, nbins16, z, 0)


def _reduce_hist(hist, histr, nbins, nbins16):
    def red(i, _):
        acc = hist[pl.ds(i * 16, 16)]
        for l in range(1, 16):
            acc = acc + hist[pl.ds(l * nbins + i * 16, 16)]
        histr[pl.ds(i * 16, 16)] = acc
        return 0
    lax.fori_loop(0, nbins16, red, 0)


def _stream_chunks(p_hbm, bufs, sems, base, per_vec):
    buf0, buf1 = bufs
    sem0, sem1 = sems

    def dma(j, buf, sem):
        return pltpu.make_async_copy(
            p_hbm.at[pl.ds(base + j * PIECE, PIECE)], buf, sem)

    def process(buf):
        def vec(v, _):
            per_vec(buf[pl.ds(v * 16, 16)])
            return 0
        lax.fori_loop(0, PIECE // 16, vec, 0, unroll=8)

    dma(0, buf0, sem0).start()

    def outer(i, _):
        j0 = 2 * i
        dma(j0 + 1, buf1, sem1).start()
        dma(j0, buf0, sem0).wait()
        process(buf0)

        @pl.when(j0 + 2 < NPIECE)
        def _():
            dma(j0 + 2, buf0, sem0).start()

        dma(j0 + 1, buf1, sem1).wait()
        process(buf1)
        return 0

    lax.fori_loop(0, NPIECE // 2, outer, 0)


def _sc_hist1(p_hbm, out_hbm, buf0, buf1, hist, histr, sem0, sem1):
    wid = _wid()
    _zero_hist(hist, 4096)
    lane_base = lax.iota(jnp.int32, 16) * 4096
    ones = jnp.ones((16,), jnp.int32)

    def per_vec(key):
        b = key >> 20
        plsc.addupdate_scatter(hist, [lane_base + b], ones)

    _stream_chunks(p_hbm, (buf0, buf1), (sem0, sem1), wid * CHUNK, per_vec)
    _reduce_hist(hist, histr, 4096, 256)
    pltpu.sync_copy(histr, out_hbm.at[wid])


def _sc_hist2(p_hbm, sel_hbm, out_hbm, buf0, buf1, hist, histr, selv, sem0, sem1):
    wid = _wid()
    _zero_hist(hist, 4096)
    pltpu.sync_copy(sel_hbm, selv)
    b1 = selv[0]
    lane_base = lax.iota(jnp.int32, 16) * 4096
    ones = jnp.ones((16,), jnp.int32)

    def per_vec(key):
        match = (key >> 20) == b1
        b = (key >> 8) & 0xFFF
        plsc.addupdate_scatter(hist, [lane_base + b], ones, mask=match)

    _stream_chunks(p_hbm, (buf0, buf1), (sem0, sem1), wid * CHUNK, per_vec)
    _reduce_hist(hist, histr, 4096, 256)
    pltpu.sync_copy(histr, out_hbm.at[wid])


def _sc_hist3(p_hbm, sel_hbm, out_hbm, buf0, buf1, hist, histr, selv, sem0, sem1):
    wid = _wid()
    _zero_hist(hist, 256)
    pltpu.sync_copy(sel_hbm, selv)
    pref = selv[0]
    lane_base = lax.iota(jnp.int32, 16) * 256
    ones = jnp.ones((16,), jnp.int32)

    def per_vec(key):
        match = (key >> 8) == pref
        b = key & 0xFF
        plsc.addupdate_scatter(hist, [lane_base + b], ones, mask=match)

    _stream_chunks(p_hbm, (buf0, buf1), (sem0, sem1), wid * CHUNK, per_vec)
    _reduce_hist(hist, histr, 256, 16)
    pltpu.sync_copy(histr, out_hbm.at[wid])


@functools.cache
def _sc_kernels():
    mesh = plsc.VectorSubcoreMesh(core_axis_name="c", subcore_axis_name="s")
    cp = pltpu.CompilerParams(needs_layout_passes=False)
    hist1 = pl.kernel(
        _sc_hist1, mesh=mesh, compiler_params=cp,
        out_type=jax.ShapeDtypeStruct((NW, 4096), jnp.int32),
        scratch_types=[pltpu.VMEM((PIECE,), jnp.int32),
                       pltpu.VMEM((PIECE,), jnp.int32),
                       pltpu.VMEM((4096 * 16,), jnp.int32),
                       pltpu.VMEM((4096,), jnp.int32),
                       pltpu.SemaphoreType.DMA,
                       pltpu.SemaphoreType.DMA])
    hist2 = pl.kernel(
        _sc_hist2, mesh=mesh, compiler_params=cp,
        out_type=jax.ShapeDtypeStruct((NW, 4096), jnp.int32),
        scratch_types=[pltpu.VMEM((PIECE,), jnp.int32),
                       pltpu.VMEM((PIECE,), jnp.int32),
                       pltpu.VMEM((4096 * 16,), jnp.int32),
                       pltpu.VMEM((4096,), jnp.int32),
                       pltpu.VMEM((2, 16), jnp.int32),
                       pltpu.SemaphoreType.DMA,
                       pltpu.SemaphoreType.DMA])
    hist3 = pl.kernel(
        _sc_hist3, mesh=mesh, compiler_params=cp,
        out_type=jax.ShapeDtypeStruct((NW, 256), jnp.int32),
        scratch_types=[pltpu.VMEM((PIECE,), jnp.int32),
                       pltpu.VMEM((PIECE,), jnp.int32),
                       pltpu.VMEM((256 * 16,), jnp.int32),
                       pltpu.VMEM((256,), jnp.int32),
                       pltpu.VMEM((2, 16), jnp.int32),
                       pltpu.SemaphoreType.DMA,
                       pltpu.SemaphoreType.DMA])
    return hist1, hist2, hist3


def _cumsum_last(x):
    n = x.shape[-1]
    s = 1
    while s < n:
        shifted = jnp.concatenate(
            [jnp.zeros_like(x[..., :s]), x[..., :n - s]], axis=-1)
        x = x + shifted
        s *= 2
    return x


def _cumsum_rows(x):
    n = x.shape[0]
    s = 1
    while s < n:
        shifted = jnp.concatenate(
            [jnp.zeros_like(x[:s]), x[:n - s]], axis=0)
        x = x + shifted
        s *= 2
    return x


def _find_bin(g, want):
    B = g.shape[1]
    cs = _cumsum_last(g)
    t = jnp.sum(g) - (cs - g)
    iota = lax.broadcasted_iota(jnp.int32, (1, B), 1)
    b = jnp.max(jnp.where(t >= want, iota, -1))
    gb = jnp.sum(jnp.where(iota == b, g, 0))
    tb = jnp.sum(jnp.where(iota == b, t, 0))
    r = want - (tb - gb)
    return b, r


def _red1_body(h_ref, o_ref):
    g = jnp.sum(h_ref[...], axis=0, keepdims=True)
    b1, r1 = _find_bin(g, KTOP)
    o_ref[0, :] = jnp.full((16,), b1, jnp.int32)
    o_ref[1, :] = jnp.full((16,), r1, jnp.int32)


def _red2_body(h_ref, s_ref, o_ref):
    g = jnp.sum(h_ref[...], axis=0, keepdims=True)
    b1 = jnp.max(s_ref[0:1, :])
    r1 = jnp.max(s_ref[1:2, :])
    b2, r2 = _find_bin(g, r1)
    o_ref[0, :] = jnp.full((16,), b1 * 4096 + b2, jnp.int32)
    o_ref[1, :] = jnp.full((16,), r2, jnp.int32)


def _red3_body(h_ref, s_ref, t_ref, r_ref):
    g = jnp.sum(h_ref[...], axis=0, keepdims=True)
    pref = jnp.max(s_ref[0:1, :])
    r2 = jnp.max(s_ref[1:2, :])
    b3, r3 = _find_bin(g, r2)
    tbits = jnp.full((1, 1), pref * 256 + b3, jnp.int32)
    t_ref[...] = lax.bitcast_convert_type(tbits, jnp.float32)
    r_ref[...] = jnp.full((1, 1), r3, jnp.int32)


def _mask_body(t_ref, r_ref, p_ref, o_ref, carry):
    pid = pl.program_id(0)

    @pl.when(pid == 0)
    def _():
        carry[0] = 0

    t = t_ref[0, 0]
    r = r_ref[0, 0]
    p = p_ref[...]
    gt = p > t
    eq = p == t
    eqi = eq.astype(jnp.int32)
    blk = jnp.sum(eqi)
    c0 = carry[0]
    take_all = (c0 + blk) <= r
    o_ref[...] = jnp.where(gt | (eq & take_all), 1.0, 0.0)

    boundary = (c0 < r) & ((c0 + blk) > r)

    @pl.when(boundary)
    def _():
        lane_cum = _cumsum_last(eqi)
        row_tot = jnp.sum(eqi, axis=1, keepdims=True)
        row_cum_excl = _cumsum_rows(row_tot) - row_tot
        rank = c0 + row_cum_excl + lane_cum
        sel = eq & (rank <= r)
        o_ref[...] = jnp.where(gt | sel, 1.0, 0.0)

    carry[0] = c0 + blk


def kernel(support_map, context_vec, dw_w, dw_b, pw_w, pw_b):
    m = jnp.mean(support_map, axis=0)
    m_bf = jnp.pad(m, ((0, 0), (1, 1), (1, 1))).astype(jnp.bfloat16)
    w3 = dw_w.reshape(C, 3, 3)
    b3 = dw_b.reshape(C, 1, 1)

    y = pl.pallas_call(
        _dw_body,
        out_shape=jax.ShapeDtypeStruct((C, H, W), jnp.float32),
        grid=(C // CB,),
        in_specs=[
            pl.BlockSpec((CB, H + 2, W + 2), lambda i: (i, 0, 0)),
            pl.BlockSpec((CB, 3, 3), lambda i: (i, 0, 0)),
            pl.BlockSpec((CB, 1, 1), lambda i: (i, 0, 0)),
        ],
        out_specs=pl.BlockSpec((CB, H, W), lambda i: (i, 0, 0)),
    )(m_bf, w3, b3)

    y2 = y.reshape(C, H * W)
    w2 = pw_w.reshape(C, C)
    b2 = pw_b.reshape(C, 1)

    p2 = pl.pallas_call(
        _pw_body,
        out_shape=jax.ShapeDtypeStruct((C, H * W), jnp.float32),
        grid=(H * W // HWT,),
        in_specs=[
            pl.BlockSpec((C, C), lambda j: (0, 0)),
            pl.BlockSpec((C, HWT), lambda j: (0, j)),
            pl.BlockSpec((C, 1), lambda j: (0, 0)),
        ],
        out_specs=pl.BlockSpec((C, HWT), lambda j: (0, j)),
    )(w2, y2, b2)

    p_lin = p2.reshape(N)
    p_bits = lax.bitcast_convert_type(p_lin, jnp.int32)

    _hist1, _hist2, _hist3 = _sc_kernels()
    h1 = _hist1(p_bits)
    sel1 = pl.pallas_call(
        _red1_body,
        out_shape=jax.ShapeDtypeStruct((2, 16), jnp.int32),
    )(h1)

    h2 = _hist2(p_bits, sel1)
    sel2 = pl.pallas_call(
        _red2_body,
        out_shape=jax.ShapeDtypeStruct((2, 16), jnp.int32),
    )(h2, sel1)

    h3 = _hist3(p_bits, sel2)
    t_arr, r_arr = pl.pallas_call(
        _red3_body,
        out_shape=(jax.ShapeDtypeStruct((1, 1), jnp.float32),
                   jax.ShapeDtypeStruct((1, 1), jnp.int32)),
    )(h3, sel2)

    p_rows = p_lin.reshape(NROWS, 128)
    mask_rows = pl.pallas_call(
        _mask_body,
        out_shape=jax.ShapeDtypeStruct((NROWS, 128), jnp.float32),
        grid=(NROWS // MROWS,),
        in_specs=[
            pl.BlockSpec(memory_space=pltpu.SMEM),
            pl.BlockSpec(memory_space=pltpu.SMEM),
            pl.BlockSpec((MROWS, 128), lambda i: (i, 0)),
        ],
        out_specs=pl.BlockSpec((MROWS, 128), lambda i: (i, 0)),
        scratch_shapes=[pltpu.SMEM((1,), jnp.int32)],
    )(t_arr, r_arr, p_rows)

    return (mask_rows.reshape(1, N), p_lin.reshape(1, N))


# speedup vs baseline: 25.3622x; 1.4731x over previous
import functools

import jax
import jax.numpy as jnp
from jax import lax
from jax.experimental import pallas as pl
from jax.experimental.pallas import tpu as pltpu
from jax.experimental.pallas import tpu_sc as plsc

C = 384
H = 224
W = 224
KTOP = 100000
N = C * H * W
NW = 32
CHUNK = N // NW
PIECE = 25088
NPIECE = CHUNK // PIECE
CB = 16
HWT = 512
NROWS = N // 128
MROWS = 1024


def _dw_body(m_ref, w_ref, b_ref, o_ref):
    m = m_ref[...].astype(jnp.float32)
    w = w_ref[...]
    acc = jnp.zeros((CB, H, W), jnp.float32)
    for ki in range(3):
        for kj in range(3):
            wv = w[:, ki, kj][:, None, None]
            acc = acc + wv * m[:, ki:ki + H, kj:kj + W]
    o_ref[...] = jnp.maximum(acc + b_ref[...], 0.0)


def _pw_body(w_ref, y_ref, b_ref, o_ref):
    lo = jnp.dot(w_ref[...].astype(jnp.bfloat16),
                 y_ref[...].astype(jnp.bfloat16),
                 preferred_element_type=jnp.float32)
    o_ref[...] = jax.nn.sigmoid(lo + b_ref[...])


def _wid():
    return lax.axis_index("s") * 2 + lax.axis_index("c")


def _zero_hist(hist, nbins16):
    def z(i, _):
        hist[pl.ds(i * 16, 16)] = jnp.zeros((16,), jnp.int32)
        return 0
    lax.fori_loop(0, nbins16, z, 0)


def _reduce_hist(hist, histr, nbins, nbins16):
    def red(i, _):
        acc = hist[pl.ds(i * 16, 16)]
        for l in range(1, 16):
            acc = acc + hist[pl.ds(l * nbins + i * 16, 16)]
        histr[pl.ds(i * 16, 16)] = acc
        return 0
    lax.fori_loop(0, nbins16, red, 0)


def _stream_chunks(p_hbm, bufs, sems, base, per_vec):
    buf0, buf1 = bufs
    sem0, sem1 = sems

    def dma(j, buf, sem):
        return pltpu.make_async_copy(
            p_hbm.at[pl.ds(base + j * PIECE, PIECE)], buf, sem)

    def process(buf):
        @plsc.parallel_loop(0, PIECE, 16, unroll=8)
        def _vec(v):
            per_vec(buf[pl.ds(v, 16)])

    dma(0, buf0, sem0).start()

    def outer(i, _):
        j0 = 2 * i
        dma(j0 + 1, buf1, sem1).start()
        dma(j0, buf0, sem0).wait()
        process(buf0)

        @pl.when(j0 + 2 < NPIECE)
        def _():
            dma(j0 + 2, buf0, sem0).start()

        dma(j0 + 1, buf1, sem1).wait()
        process(buf1)
        return 0

    lax.fori_loop(0, NPIECE // 2, outer, 0)


def _sc_hist1(p_hbm, out_hbm, buf0, buf1, hist, histr, sem0, sem1):
    wid = _wid()
    _zero_hist(hist, 4096)
    lane_base = lax.iota(jnp.int32, 16) * 4096
    ones = jnp.ones((16,), jnp.int32)

    def per_vec(key):
        b = key >> 20
        plsc.addupdate_scatter(hist, [lane_base + b], ones)

    _stream_chunks(p_hbm, (buf0, buf1), (sem0, sem1), wid * CHUNK, per_vec)
    _reduce_hist(hist, histr, 4096, 256)
    pltpu.sync_copy(histr, out_hbm.at[wid])


def _sc_hist2(p_hbm, sel_hbm, out_hbm, buf0, buf1, hist, histr, selv, sem0, sem1):
    wid = _wid()
    _zero_hist(hist, 4096)
    pltpu.sync_copy(sel_hbm, selv)
    b1 = selv[0]
    lane_base = lax.iota(jnp.int32, 16) * 4096
    ones = jnp.ones((16,), jnp.int32)

    def per_vec(key):
        match = (key >> 20) == b1
        b = (key >> 8) & 0xFFF
        plsc.addupdate_scatter(hist, [lane_base + b], ones, mask=match)

    _stream_chunks(p_hbm, (buf0, buf1), (sem0, sem1), wid * CHUNK, per_vec)
    _reduce_hist(hist, histr, 4096, 256)
    pltpu.sync_copy(histr, out_hbm.at[wid])


def _sc_hist3(p_hbm, sel_hbm, out_hbm, buf0, buf1, hist, histr, selv, sem0, sem1):
    wid = _wid()
    _zero_hist(hist, 256)
    pltpu.sync_copy(sel_hbm, selv)
    pref = selv[0]
    lane_base = lax.iota(jnp.int32, 16) * 256
    ones = jnp.ones((16,), jnp.int32)

    def per_vec(key):
        match = (key >> 8) == pref
        b = key & 0xFF
        plsc.addupdate_scatter(hist, [lane_base + b], ones, mask=match)

    _stream_chunks(p_hbm, (buf0, buf1), (sem0, sem1), wid * CHUNK, per_vec)
    _reduce_hist(hist, histr, 256, 16)
    pltpu.sync_copy(histr, out_hbm.at[wid])


@functools.cache
def _sc_kernels():
    mesh = plsc.VectorSubcoreMesh(core_axis_name="c", subcore_axis_name="s")
    cp = pltpu.CompilerParams(needs_layout_passes=False)
    hist1 = pl.kernel(
        _sc_hist1, mesh=mesh, compiler_params=cp,
        out_type=jax.ShapeDtypeStruct((NW, 4096), jnp.int32),
        scratch_types=[pltpu.VMEM((PIECE,), jnp.int32),
                       pltpu.VMEM((PIECE,), jnp.int32),
                       pltpu.VMEM((4096 * 16,), jnp.int32),
                       pltpu.VMEM((4096,), jnp.int32),
                       pltpu.SemaphoreType.DMA,
                       pltpu.SemaphoreType.DMA])
    hist2 = pl.kernel(
        _sc_hist2, mesh=mesh, compiler_params=cp,
        out_type=jax.ShapeDtypeStruct((NW, 4096), jnp.int32),
        scratch_types=[pltpu.VMEM((PIECE,), jnp.int32),
                       pltpu.VMEM((PIECE,), jnp.int32),
                       pltpu.VMEM((4096 * 16,), jnp.int32),
                       pltpu.VMEM((4096,), jnp.int32),
                       pltpu.VMEM((2, 16), jnp.int32),
                       pltpu.SemaphoreType.DMA,
                       pltpu.SemaphoreType.DMA])
    hist3 = pl.kernel(
        _sc_hist3, mesh=mesh, compiler_params=cp,
        out_type=jax.ShapeDtypeStruct((NW, 256), jnp.int32),
        scratch_types=[pltpu.VMEM((PIECE,), jnp.int32),
                       pltpu.VMEM((PIECE,), jnp.int32),
                       pltpu.VMEM((256 * 16,), jnp.int32),
                       pltpu.VMEM((256,), jnp.int32),
                       pltpu.VMEM((2, 16), jnp.int32),
                       pltpu.SemaphoreType.DMA,
                       pltpu.SemaphoreType.DMA])
    return hist1, hist2, hist3


def _cumsum_last(x):
    n = x.shape[-1]
    s = 1
    while s < n:
        shifted = jnp.concatenate(
            [jnp.zeros_like(x[..., :s]), x[..., :n - s]], axis=-1)
        x = x + shifted
        s *= 2
    return x


def _cumsum_rows(x):
    n = x.shape[0]
    s = 1
    while s < n:
        shifted = jnp.concatenate(
            [jnp.zeros_like(x[:s]), x[:n - s]], axis=0)
        x = x + shifted
        s *= 2
    return x


def _find_bin(g, want):
    B = g.shape[1]
    cs = _cumsum_last(g)
    t = jnp.sum(g) - (cs - g)
    iota = lax.broadcasted_iota(jnp.int32, (1, B), 1)
    b = jnp.max(jnp.where(t >= want, iota, -1))
    gb = jnp.sum(jnp.where(iota == b, g, 0))
    tb = jnp.sum(jnp.where(iota == b, t, 0))
    r = want - (tb - gb)
    return b, r


def _red1_body(h_ref, o_ref):
    g = jnp.sum(h_ref[...], axis=0, keepdims=True)
    b1, r1 = _find_bin(g, KTOP)
    o_ref[0, :] = jnp.full((16,), b1, jnp.int32)
    o_ref[1, :] = jnp.full((16,), r1, jnp.int32)


def _red2_body(h_ref, s_ref, o_ref):
    g = jnp.sum(h_ref[...], axis=0, keepdims=True)
    b1 = jnp.max(s_ref[0:1, :])
    r1 = jnp.max(s_ref[1:2, :])
    b2, r2 = _find_bin(g, r1)
    o_ref[0, :] = jnp.full((16,), b1 * 4096 + b2, jnp.int32)
    o_ref[1, :] = jnp.full((16,), r2, jnp.int32)


def _red3_body(h_ref, s_ref, t_ref, r_ref):
    g = jnp.sum(h_ref[...], axis=0, keepdims=True)
    pref = jnp.max(s_ref[0:1, :])
    r2 = jnp.max(s_ref[1:2, :])
    b3, r3 = _find_bin(g, r2)
    tbits = jnp.full((1, 1), pref * 256 + b3, jnp.int32)
    t_ref[...] = lax.bitcast_convert_type(tbits, jnp.float32)
    r_ref[...] = jnp.full((1, 1), r3, jnp.int32)


def _mask_body(t_ref, r_ref, p_ref, o_ref, carry):
    pid = pl.program_id(0)

    @pl.when(pid == 0)
    def _():
        carry[0] = 0

    t = t_ref[0, 0]
    r = r_ref[0, 0]
    p = p_ref[...]
    gt = p > t
    eq = p == t
    eqi = eq.astype(jnp.int32)
    blk = jnp.sum(eqi)
    c0 = carry[0]
    take_all = (c0 + blk) <= r
    o_ref[...] = jnp.where(gt | (eq & take_all), 1.0, 0.0)

    boundary = (c0 < r) & ((c0 + blk) > r)

    @pl.when(boundary)
    def _():
        lane_cum = _cumsum_last(eqi)
        row_tot = jnp.sum(eqi, axis=1, keepdims=True)
        row_cum_excl = _cumsum_rows(row_tot) - row_tot
        rank = c0 + row_cum_excl + lane_cum
        sel = eq & (rank <= r)
        o_ref[...] = jnp.where(gt | sel, 1.0, 0.0)

    carry[0] = c0 + blk


def kernel(support_map, context_vec, dw_w, dw_b, pw_w, pw_b):
    m = jnp.mean(support_map, axis=0)
    m_bf = jnp.pad(m, ((0, 0), (1, 1), (1, 1))).astype(jnp.bfloat16)
    w3 = dw_w.reshape(C, 3, 3)
    b3 = dw_b.reshape(C, 1, 1)

    y = pl.pallas_call(
        _dw_body,
        out_shape=jax.ShapeDtypeStruct((C, H, W), jnp.float32),
        grid=(C // CB,),
        in_specs=[
            pl.BlockSpec((CB, H + 2, W + 2), lambda i: (i, 0, 0)),
            pl.BlockSpec((CB, 3, 3), lambda i: (i, 0, 0)),
            pl.BlockSpec((CB, 1, 1), lambda i: (i, 0, 0)),
        ],
        out_specs=pl.BlockSpec((CB, H, W), lambda i: (i, 0, 0)),
    )(m_bf, w3, b3)

    y2 = y.reshape(C, H * W)
    w2 = pw_w.reshape(C, C)
    b2 = pw_b.reshape(C, 1)

    p2 = pl.pallas_call(
        _pw_body,
        out_shape=jax.ShapeDtypeStruct((C, H * W), jnp.float32),
        grid=(H * W // HWT,),
        in_specs=[
            pl.BlockSpec((C, C), lambda j: (0, 0)),
            pl.BlockSpec((C, HWT), lambda j: (0, j)),
            pl.BlockSpec((C, 1), lambda j: (0, 0)),
        ],
        out_specs=pl.BlockSpec((C, HWT), lambda j: (0, j)),
    )(w2, y2, b2)

    p_lin = p2.reshape(N)
    p_bits = lax.bitcast_convert_type(p_lin, jnp.int32)

    _hist1, _hist2, _hist3 = _sc_kernels()
    h1 = _hist1(p_bits)
    sel1 = pl.pallas_call(
        _red1_body,
        out_shape=jax.ShapeDtypeStruct((2, 16), jnp.int32),
    )(h1)

    h2 = _hist2(p_bits, sel1)
    sel2 = pl.pallas_call(
        _red2_body,
        out_shape=jax.ShapeDtypeStruct((2, 16), jnp.int32),
    )(h2, sel1)

    h3 = _hist3(p_bits, sel2)
    t_arr, r_arr = pl.pallas_call(
        _red3_body,
        out_shape=(jax.ShapeDtypeStruct((1, 1), jnp.float32),
                   jax.ShapeDtypeStruct((1, 1), jnp.int32)),
    )(h3, sel2)

    p_rows = p_lin.reshape(NROWS, 128)
    mask_rows = pl.pallas_call(
        _mask_body,
        out_shape=jax.ShapeDtypeStruct((NROWS, 128), jnp.float32),
        grid=(NROWS // MROWS,),
        in_specs=[
            pl.BlockSpec(memory_space=pltpu.SMEM),
            pl.BlockSpec(memory_space=pltpu.SMEM),
            pl.BlockSpec((MROWS, 128), lambda i: (i, 0)),
        ],
        out_specs=pl.BlockSpec((MROWS, 128), lambda i: (i, 0)),
        scratch_shapes=[pltpu.SMEM((1,), jnp.int32)],
    )(t_arr, r_arr, p_rows)

    return (mask_rows.reshape(1, N), p_lin.reshape(1, N))


# speedup vs baseline: 26.9920x; 1.0643x over previous
import functools

import jax
import jax.numpy as jnp
from jax import lax
from jax.experimental import pallas as pl
from jax.experimental.pallas import tpu as pltpu
from jax.experimental.pallas import tpu_sc as plsc

C = 384
H = 224
W = 224
KTOP = 100000
N = C * H * W
NW = 32
CHUNK = N // NW
PIECE = 25088
NPIECE = CHUNK // PIECE
CB = 16
HWT = 512
NROWS = N // 128
MROWS = 1024


def _dw_body(x_ref, w_ref, b_ref, o_ref):
    m32 = 0.5 * (x_ref[0] + x_ref[1])
    m = m32.astype(jnp.bfloat16).astype(jnp.float32)
    z_r = jnp.zeros((CB, 1, W), jnp.float32)
    mp = jnp.concatenate([z_r, m, z_r], axis=1)
    z_c = jnp.zeros((CB, H + 2, 1), jnp.float32)
    mp = jnp.concatenate([z_c, mp, z_c], axis=2)
    w = w_ref[...]
    acc = jnp.zeros((CB, H, W), jnp.float32)
    for ki in range(3):
        for kj in range(3):
            wv = w[:, ki, kj][:, None, None]
            acc = acc + wv * mp[:, ki:ki + H, kj:kj + W]
    o_ref[...] = jnp.maximum(acc + b_ref[...], 0.0).astype(jnp.bfloat16)


def _pw_body(w_ref, y_ref, b_ref, o_ref):
    lo = jnp.dot(w_ref[...].astype(jnp.bfloat16), y_ref[...],
                 preferred_element_type=jnp.float32)
    o_ref[...] = jax.nn.sigmoid(lo + b_ref[...])


def _wid():
    return lax.axis_index("s") * 2 + lax.axis_index("c")


def _zero_hist(hist, nbins16):
    def z(i, _):
        hist[pl.ds(i * 16, 16)] = jnp.zeros((16,), jnp.int32)
        return 0
    lax.fori_loop(0, nbins16, z, 0)


def _reduce_hist(hist, histr, nbins, nbins16):
    def red(i, _):
        acc = hist[pl.ds(i * 16, 16)]
        for l in range(1, 16):
            acc = acc + hist[pl.ds(l * nbins + i * 16, 16)]
        histr[pl.ds(i * 16, 16)] = acc
        return 0
    lax.fori_loop(0, nbins16, red, 0)


def _stream_chunks(p_hbm, bufs, sems, base, per_vec):
    buf0, buf1 = bufs
    sem0, sem1 = sems

    def dma(j, buf, sem):
        return pltpu.make_async_copy(
            p_hbm.at[pl.ds(base + j * PIECE, PIECE)], buf, sem)

    def process(buf):
        @plsc.parallel_loop(0, PIECE, 16, unroll=8)
        def _vec(v):
            per_vec(buf[pl.ds(v, 16)])

    dma(0, buf0, sem0).start()

    def outer(i, _):
        j0 = 2 * i
        dma(j0 + 1, buf1, sem1).start()
        dma(j0, buf0, sem0).wait()
        process(buf0)

        @pl.when(j0 + 2 < NPIECE)
        def _():
            dma(j0 + 2, buf0, sem0).start()

        dma(j0 + 1, buf1, sem1).wait()
        process(buf1)
        return 0

    lax.fori_loop(0, NPIECE // 2, outer, 0)


def _sc_hist1(p_hbm, out_hbm, buf0, buf1, hist, histr, sem0, sem1):
    wid = _wid()
    _zero_hist(hist, 4096)
    lane_base = lax.iota(jnp.int32, 16) * 4096
    ones = jnp.ones((16,), jnp.int32)

    def per_vec(key):
        b = key >> 20
        plsc.addupdate_scatter(hist, [lane_base + b], ones)

    _stream_chunks(p_hbm, (buf0, buf1), (sem0, sem1), wid * CHUNK, per_vec)
    _reduce_hist(hist, histr, 4096, 256)
    pltpu.sync_copy(histr, out_hbm.at[wid])


def _sc_hist2(p_hbm, sel_hbm, out_hbm, buf0, buf1, hist, histr, selv, sem0, sem1):
    wid = _wid()
    _zero_hist(hist, 4096)
    pltpu.sync_copy(sel_hbm, selv)
    b1 = selv[0]
    lane_base = lax.iota(jnp.int32, 16) * 4096
    ones = jnp.ones((16,), jnp.int32)

    def per_vec(key):
        match = (key >> 20) == b1
        b = (key >> 8) & 0xFFF
        plsc.addupdate_scatter(hist, [lane_base + b], ones, mask=match)

    _stream_chunks(p_hbm, (buf0, buf1), (sem0, sem1), wid * CHUNK, per_vec)
    _reduce_hist(hist, histr, 4096, 256)
    pltpu.sync_copy(histr, out_hbm.at[wid])


def _sc_hist3(p_hbm, sel_hbm, out_hbm, buf0, buf1, hist, histr, selv, sem0, sem1):
    wid = _wid()
    _zero_hist(hist, 256)
    pltpu.sync_copy(sel_hbm, selv)
    pref = selv[0]
    lane_base = lax.iota(jnp.int32, 16) * 256
    ones = jnp.ones((16,), jnp.int32)

    def per_vec(key):
        match = (key >> 8) == pref
        b = key & 0xFF
        plsc.addupdate_scatter(hist, [lane_base + b], ones, mask=match)

    _stream_chunks(p_hbm, (buf0, buf1), (sem0, sem1), wid * CHUNK, per_vec)
    _reduce_hist(hist, histr, 256, 16)
    pltpu.sync_copy(histr, out_hbm.at[wid])


@functools.cache
def _sc_kernels():
    mesh = plsc.VectorSubcoreMesh(core_axis_name="c", subcore_axis_name="s")
    cp = pltpu.CompilerParams(needs_layout_passes=False)
    hist1 = pl.kernel(
        _sc_hist1, mesh=mesh, compiler_params=cp,
        out_type=jax.ShapeDtypeStruct((NW, 4096), jnp.int32),
        scratch_types=[pltpu.VMEM((PIECE,), jnp.int32),
                       pltpu.VMEM((PIECE,), jnp.int32),
                       pltpu.VMEM((4096 * 16,), jnp.int32),
                       pltpu.VMEM((4096,), jnp.int32),
                       pltpu.SemaphoreType.DMA,
                       pltpu.SemaphoreType.DMA])
    hist2 = pl.kernel(
        _sc_hist2, mesh=mesh, compiler_params=cp,
        out_type=jax.ShapeDtypeStruct((NW, 4096), jnp.int32),
        scratch_types=[pltpu.VMEM((PIECE,), jnp.int32),
                       pltpu.VMEM((PIECE,), jnp.int32),
                       pltpu.VMEM((4096 * 16,), jnp.int32),
                       pltpu.VMEM((4096,), jnp.int32),
                       pltpu.VMEM((2, 16), jnp.int32),
                       pltpu.SemaphoreType.DMA,
                       pltpu.SemaphoreType.DMA])
    hist3 = pl.kernel(
        _sc_hist3, mesh=mesh, compiler_params=cp,
        out_type=jax.ShapeDtypeStruct((NW, 256), jnp.int32),
        scratch_types=[pltpu.VMEM((PIECE,), jnp.int32),
                       pltpu.VMEM((PIECE,), jnp.int32),
                       pltpu.VMEM((256 * 16,), jnp.int32),
                       pltpu.VMEM((256,), jnp.int32),
                       pltpu.VMEM((2, 16), jnp.int32),
                       pltpu.SemaphoreType.DMA,
                       pltpu.SemaphoreType.DMA])
    return hist1, hist2, hist3


def _cumsum_last(x):
    n = x.shape[-1]
    s = 1
    while s < n:
        shifted = jnp.concatenate(
            [jnp.zeros_like(x[..., :s]), x[..., :n - s]], axis=-1)
        x = x + shifted
        s *= 2
    return x


def _cumsum_rows(x):
    n = x.shape[0]
    s = 1
    while s < n:
        shifted = jnp.concatenate(
            [jnp.zeros_like(x[:s]), x[:n - s]], axis=0)
        x = x + shifted
        s *= 2
    return x


def _find_bin(g, want):
    B = g.shape[1]
    cs = _cumsum_last(g)
    t = jnp.sum(g) - (cs - g)
    iota = lax.broadcasted_iota(jnp.int32, (1, B), 1)
    b = jnp.max(jnp.where(t >= want, iota, -1))
    gb = jnp.sum(jnp.where(iota == b, g, 0))
    tb = jnp.sum(jnp.where(iota == b, t, 0))
    r = want - (tb - gb)
    return b, r


def _red1_body(h_ref, o_ref):
    g = jnp.sum(h_ref[...], axis=0, keepdims=True)
    b1, r1 = _find_bin(g, KTOP)
    o_ref[0, :] = jnp.full((16,), b1, jnp.int32)
    o_ref[1, :] = jnp.full((16,), r1, jnp.int32)


def _red2_body(h_ref, s_ref, o_ref):
    g = jnp.sum(h_ref[...], axis=0, keepdims=True)
    b1 = jnp.max(s_ref[0:1, :])
    r1 = jnp.max(s_ref[1:2, :])
    b2, r2 = _find_bin(g, r1)
    o_ref[0, :] = jnp.full((16,), b1 * 4096 + b2, jnp.int32)
    o_ref[1, :] = jnp.full((16,), r2, jnp.int32)


def _red3_body(h_ref, s_ref, t_ref, r_ref):
    g = jnp.sum(h_ref[...], axis=0, keepdims=True)
    pref = jnp.max(s_ref[0:1, :])
    r2 = jnp.max(s_ref[1:2, :])
    b3, r3 = _find_bin(g, r2)
    tbits = jnp.full((1, 1), pref * 256 + b3, jnp.int32)
    t_ref[...] = lax.bitcast_convert_type(tbits, jnp.float32)
    r_ref[...] = jnp.full((1, 1), r3, jnp.int32)


def _mask_body(t_ref, r_ref, p_ref, o_ref, carry):
    pid = pl.program_id(0)

    @pl.when(pid == 0)
    def _():
        carry[0] = 0

    t = t_ref[0, 0]
    r = r_ref[0, 0]
    p = p_ref[...]
    gt = p > t
    eq = p == t
    eqi = eq.astype(jnp.int32)
    blk = jnp.sum(eqi)
    c0 = carry[0]
    take_all = (c0 + blk) <= r
    o_ref[...] = jnp.where(gt | (eq & take_all), 1.0, 0.0)

    boundary = (c0 < r) & ((c0 + blk) > r)

    @pl.when(boundary)
    def _():
        lane_cum = _cumsum_last(eqi)
        row_tot = jnp.sum(eqi, axis=1, keepdims=True)
        row_cum_excl = _cumsum_rows(row_tot) - row_tot
        rank = c0 + row_cum_excl + lane_cum
        sel = eq & (rank <= r)
        o_ref[...] = jnp.where(gt | sel, 1.0, 0.0)

    carry[0] = c0 + blk


def kernel(support_map, context_vec, dw_w, dw_b, pw_w, pw_b):
    w3 = dw_w.reshape(C, 3, 3)
    b3 = dw_b.reshape(C, 1, 1)

    y = pl.pallas_call(
        _dw_body,
        out_shape=jax.ShapeDtypeStruct((C, H, W), jnp.bfloat16),
        grid=(C // CB,),
        in_specs=[
            pl.BlockSpec((2, CB, H, W), lambda i: (0, i, 0, 0)),
            pl.BlockSpec((CB, 3, 3), lambda i: (i, 0, 0)),
            pl.BlockSpec((CB, 1, 1), lambda i: (i, 0, 0)),
        ],
        out_specs=pl.BlockSpec((CB, H, W), lambda i: (i, 0, 0)),
    )(support_map, w3, b3)

    y2 = y.reshape(C, H * W)
    w2 = pw_w.reshape(C, C)
    b2 = pw_b.reshape(C, 1)

    p2 = pl.pallas_call(
        _pw_body,
        out_shape=jax.ShapeDtypeStruct((C, H * W), jnp.float32),
        grid=(H * W // HWT,),
        in_specs=[
            pl.BlockSpec((C, C), lambda j: (0, 0)),
            pl.BlockSpec((C, HWT), lambda j: (0, j)),
            pl.BlockSpec((C, 1), lambda j: (0, 0)),
        ],
        out_specs=pl.BlockSpec((C, HWT), lambda j: (0, j)),
    )(w2, y2, b2)

    p_lin = p2.reshape(N)
    p_bits = lax.bitcast_convert_type(p_lin, jnp.int32)

    _hist1, _hist2, _hist3 = _sc_kernels()
    h1 = _hist1(p_bits)
    sel1 = pl.pallas_call(
        _red1_body,
        out_shape=jax.ShapeDtypeStruct((2, 16), jnp.int32),
    )(h1)

    h2 = _hist2(p_bits, sel1)
    sel2 = pl.pallas_call(
        _red2_body,
        out_shape=jax.ShapeDtypeStruct((2, 16), jnp.int32),
    )(h2, sel1)

    h3 = _hist3(p_bits, sel2)
    t_arr, r_arr = pl.pallas_call(
        _red3_body,
        out_shape=(jax.ShapeDtypeStruct((1, 1), jnp.float32),
                   jax.ShapeDtypeStruct((1, 1), jnp.int32)),
    )(h3, sel2)

    p_rows = p_lin.reshape(NROWS, 128)
    mask_rows = pl.pallas_call(
        _mask_body,
        out_shape=jax.ShapeDtypeStruct((NROWS, 128), jnp.float32),
        grid=(NROWS // MROWS,),
        in_specs=[
            pl.BlockSpec(memory_space=pltpu.SMEM),
            pl.BlockSpec(memory_space=pltpu.SMEM),
            pl.BlockSpec((MROWS, 128), lambda i: (i, 0)),
        ],
        out_specs=pl.BlockSpec((MROWS, 128), lambda i: (i, 0)),
        scratch_shapes=[pltpu.SMEM((1,), jnp.int32)],
    )(t_arr, r_arr, p_rows)

    return (mask_rows.reshape(1, N), p_lin.reshape(1, N))
